# baseline (speedup 1.0000x reference)
import functools

import jax
import jax.numpy as jnp
from jax import lax
from jax.experimental import pallas as pl
from jax.experimental.pallas import tpu as pltpu
from jax.experimental.pallas import tpu_sc as plsc

N_NODES = 10000
NUM_EDGES = 320000
NC = 2
NS = 16
CH = 128
TOTAL_CH = 2560
E_PAD = TOTAL_CH * CH
N_PAD = 10240
ZR = N_PAD // NS
ROWS_BLK = 1000
GRID = N_NODES // ROWS_BLK

_MESH = plsc.VectorSubcoreMesh(core_axis_name="c", subcore_axis_name="s")
_SC_PARAMS = pltpu.CompilerParams(use_tc_tiling_on_sc=False)


def _zero_buf(buf, rows, cols):
    @pl.loop(0, rows)
    def _(r):
        @pl.loop(0, cols // 16)
        def _(cc):
            buf[r, pl.ds(cc * 16, 16)] = jnp.zeros((16,), jnp.float32)


def _fill_ones(buf, rows, cols):
    @pl.loop(0, rows)
    def _(r):
        @pl.loop(0, cols // 16)
        def _(cc):
            buf[r, pl.ds(cc * 16, 16)] = jnp.ones((16,), jnp.float32)


def _tile_span(cid, sid, c0, c1):
    return jnp.where(cid == 0, sid * c0, NS * c0 + sid * c1)


def _zero_acc(buf, acc, sid):
    @pl.loop(0, ZR // CH)
    def _(r):
        pltpu.sync_copy(buf, acc.at[pl.ds(sid * ZR + r * CH, CH)])


def _copy_out(acc, out_hbm, cid, sid):
    oc = out_hbm.at[cid]

    @pl.loop(0, ZR // CH)
    def _(r):
        base = sid * ZR + r * CH
        pltpu.sync_copy(acc.at[pl.ds(base, CH)], oc.at[pl.ds(base, CH)])


def _sc_degree(dstc, c0, c1):
    nmax = max(c0, c1)

    @functools.partial(
        pl.kernel,
        out_type=jax.ShapeDtypeStruct((NC, N_PAD, 16), jnp.float32),
        mesh=_MESH,
        scratch_types=[
            pltpu.VMEM((nmax, CH), jnp.int32),
            pltpu.VMEM((CH, 16), jnp.float32),
            pltpu.VMEM_SHARED((N_PAD, 16), jnp.float32),
        ],
        compiler_params=_SC_PARAMS,
    )
    def k(dst_hbm, out_hbm, didx, buf, acc):
        cid = lax.axis_index("c")
        sid = lax.axis_index("s")
        start = _tile_span(cid, sid, c0, c1)
        lbase = jnp.minimum(start, TOTAL_CH - nmax)
        off = start - lbase
        pltpu.sync_copy(dst_hbm.at[pl.ds(lbase, nmax)], didx)
        _zero_buf(buf, CH, 16)
        _zero_acc(buf, acc, sid)
        _fill_ones(buf, CH, 16)
        plsc.subcore_barrier()

        def run(cnt):
            @pl.loop(0, cnt)
            def _(j):
                pltpu.sync_copy(buf, acc.at[didx.at[off + j]], add=True)

        @pl.when(cid == 0)
        def _():
            run(c0)

        @pl.when(cid == 1)
        def _():
            run(c1)

        plsc.subcore_barrier()
        _copy_out(acc, out_hbm, cid, sid)

    return k(dstc)


NBUF = 8


def _sc_agg(y, srcc, dstc, feat, c0, c1):
    nmax = max(c0, c1)
    assert c0 % NBUF == 0 and c1 % NBUF == 0

    @functools.partial(
        pl.kernel,
        out_type=jax.ShapeDtypeStruct((NC, N_PAD, feat), jnp.float32),
        mesh=_MESH,
        scratch_types=(
            [pltpu.VMEM((nmax, CH), jnp.int32), pltpu.VMEM((nmax, CH), jnp.int32)]
            + [pltpu.VMEM((CH, feat), jnp.float32) for _ in range(NBUF)]
            + [pltpu.VMEM_SHARED((N_PAD, feat), jnp.float32)]
            + [pltpu.SemaphoreType.DMA for _ in range(NBUF)]
        ),
        compiler_params=_SC_PARAMS,
    )
    def k(y_hbm, src_hbm, dst_hbm, out_hbm, sidx, didx, *rest):
        bufs = rest[:NBUF]
        acc = rest[NBUF]
        sems = rest[NBUF + 1:]
        cid = lax.axis_index("c")
        sid = lax.axis_index("s")
        start = _tile_span(cid, sid, c0, c1)
        lbase = jnp.minimum(start, TOTAL_CH - nmax)
        off = start - lbase
        pltpu.sync_copy(src_hbm.at[pl.ds(lbase, nmax)], sidx)
        pltpu.sync_copy(dst_hbm.at[pl.ds(lbase, nmax)], didx)
        _zero_buf(bufs[0], CH, feat)
        _zero_acc(bufs[0], acc, sid)
        plsc.subcore_barrier()

        def run(cnt):
            for b in range(NBUF):
                pltpu.make_async_copy(y_hbm.at[sidx.at[off + b]], bufs[b], sems[b]).start()

            @pl.loop(0, cnt // NBUF)
            def _(g):
                base = off + g * NBUF
                for b in range(NBUF):
                    j = base + b
                    pltpu.make_async_copy(y_hbm.at[sidx.at[j]], bufs[b], sems[b]).wait()
                    pltpu.sync_copy(bufs[b], acc.at[didx.at[j]], add=True)

                    @pl.when(j + NBUF < off + cnt)
                    def _():
                        pltpu.make_async_copy(
                            y_hbm.at[sidx.at[j + NBUF]], bufs[b], sems[b]
                        ).start()

        @pl.when(cid == 0)
        def _():
            run(c0)

        @pl.when(cid == 1)
        def _():
            run(c1)

        plsc.subcore_barrier()
        _copy_out(acc, out_hbm, cid, sid)

    return k(y, srcc, dstc)


def _dinv_from(dg_ref):
    deg = dg_ref[0, :, 0] + dg_ref[1, :, 0] + 1.0
    return lax.rsqrt(deg)


def _tc_first(x, W1, dg):
    def body(x_ref, w_ref, dg_ref, y_ref):
        dinv = _dinv_from(dg_ref)
        y_ref[...] = (
            jnp.dot(x_ref[...], w_ref[...], preferred_element_type=jnp.float32)
            * dinv[:, None]
        )

    return pl.pallas_call(
        body,
        grid=(GRID,),
        in_specs=[
            pl.BlockSpec((ROWS_BLK, 128), lambda i: (i, 0)),
            pl.BlockSpec((128, 64), lambda i: (0, 0)),
            pl.BlockSpec((NC, ROWS_BLK, 16), lambda i: (0, i, 0)),
        ],
        out_specs=pl.BlockSpec((ROWS_BLK, 64), lambda i: (i, 0)),
        out_shape=jax.ShapeDtypeStruct((N_NODES, 64), jnp.float32),
    )(x, W1, dg)


def _tc_mid(p, y_prev, dg, b, W, f_in, f_out):
    def body(p_ref, y_ref, dg_ref, b_ref, w_ref, o_ref):
        dinv = _dinv_from(dg_ref)
        h = dinv[:, None] * (p_ref[0] + p_ref[1] + y_ref[...]) + b_ref[...]
        h = jnp.maximum(h, 0.0)
        o_ref[...] = (
            jnp.dot(h, w_ref[...], preferred_element_type=jnp.float32)
            * dinv[:, None]
        )

    return pl.pallas_call(
        body,
        grid=(GRID,),
        in_specs=[
            pl.BlockSpec((NC, ROWS_BLK, f_in), lambda i: (0, i, 0)),
            pl.BlockSpec((ROWS_BLK, f_in), lambda i: (i, 0)),
            pl.BlockSpec((NC, ROWS_BLK, 16), lambda i: (0, i, 0)),
            pl.BlockSpec((1, f_in), lambda i: (0, 0)),
            pl.BlockSpec((f_in, f_out), lambda i: (0, 0)),
        ],
        out_specs=pl.BlockSpec((ROWS_BLK, f_out), lambda i: (i, 0)),
        out_shape=jax.ShapeDtypeStruct((N_NODES, f_out), jnp.float32),
    )(p, y_prev, dg, b, W)


def _tc_final(p, y3, dg, b3):
    def body(p_ref, y_ref, dg_ref, b_ref, o_ref):
        dinv = _dinv_from(dg_ref)
        o3 = (dinv[:, None] * (p_ref[0] + p_ref[1] + y_ref[...]))[:, :2] + b_ref[...]
        o_ref[...] = jax.nn.log_softmax(o3, axis=1)

    return pl.pallas_call(
        body,
        grid=(GRID,),
        in_specs=[
            pl.BlockSpec((NC, ROWS_BLK, 16), lambda i: (0, i, 0)),
            pl.BlockSpec((ROWS_BLK, 16), lambda i: (i, 0)),
            pl.BlockSpec((NC, ROWS_BLK, 16), lambda i: (0, i, 0)),
            pl.BlockSpec((1, 2), lambda i: (0, 0)),
        ],
        out_specs=pl.BlockSpec((ROWS_BLK, 2), lambda i: (i, 0)),
        out_shape=jax.ShapeDtypeStruct((N_NODES, 2), jnp.float32),
    )(p, y3, dg, b3)


SPLIT_DEG = (80, 80)
SPLIT_L1 = (80, 80)
SPLIT_L2 = (80, 80)
SPLIT_L3 = (80, 80)


def kernel(x, edge_index, W1, b1, W2, b2, W3, b3):
    pad = E_PAD - NUM_EDGES
    src = jnp.concatenate([edge_index[0], jnp.zeros((pad,), jnp.int32)])
    trash = N_NODES + (jnp.arange(pad, dtype=jnp.int32) % (N_PAD - N_NODES))
    dst = jnp.concatenate([edge_index[1], trash])
    srcc = src.reshape(TOTAL_CH, CH)
    dstc = dst.reshape(TOTAL_CH, CH)

    dg = _sc_degree(dstc, *SPLIT_DEG)
    y1 = _tc_first(x, W1, dg)
    p1 = _sc_agg(y1, srcc, dstc, 64, *SPLIT_L1)
    y2 = _tc_mid(p1, y1, dg, b1.reshape(1, 64), W2, 64, 32)
    p2 = _sc_agg(y2, srcc, dstc, 32, *SPLIT_L2)
    W3p = jnp.zeros((32, 16), jnp.float32).at[:, :2].set(W3)
    y3 = _tc_mid(p2, y2, dg, b2.reshape(1, 32), W3p, 32, 16)
    p3 = _sc_agg(y3, srcc, dstc, 16, *SPLIT_L3)
    return _tc_final(p3, y3, dg, b3.reshape(1, 2))


# speedup vs baseline: 1.0694x; 1.0694x over previous
import functools

import jax
import jax.numpy as jnp
from jax import lax
from jax.experimental import pallas as pl
from jax.experimental.pallas import tpu as pltpu
from jax.experimental.pallas import tpu_sc as plsc

N_NODES = 10000
NUM_EDGES = 320000
NC = 2
NS = 16
CH = 128
TOTAL_CH = 2560
E_PAD = TOTAL_CH * CH
N_PAD = 10240
ZR = N_PAD // NS
ROWS_BLK = 1000
GRID = N_NODES // ROWS_BLK

_MESH = plsc.VectorSubcoreMesh(core_axis_name="c", subcore_axis_name="s")
_SC_PARAMS = pltpu.CompilerParams(use_tc_tiling_on_sc=False)


def _zero_buf(buf, rows, cols):
    @pl.loop(0, rows)
    def _(r):
        @pl.loop(0, cols // 16)
        def _(cc):
            buf[r, pl.ds(cc * 16, 16)] = jnp.zeros((16,), jnp.float32)


def _fill_ones(buf, rows, cols):
    @pl.loop(0, rows)
    def _(r):
        @pl.loop(0, cols // 16)
        def _(cc):
            buf[r, pl.ds(cc * 16, 16)] = jnp.ones((16,), jnp.float32)


def _tile_span(cid, sid, c0, c1):
    return jnp.where(cid == 0, sid * c0, NS * c0 + sid * c1)


def _zero_acc(buf, acc, sid):
    @pl.loop(0, ZR // CH)
    def _(r):
        pltpu.sync_copy(buf, acc.at[pl.ds(sid * ZR + r * CH, CH)])


def _copy_out(acc, out_hbm, cid, sid):
    oc = out_hbm.at[cid]

    @pl.loop(0, ZR // CH)
    def _(r):
        base = sid * ZR + r * CH
        pltpu.sync_copy(acc.at[pl.ds(base, CH)], oc.at[pl.ds(base, CH)])


def _sc_degree(dstc, c0, c1):
    nmax = max(c0, c1)

    @functools.partial(
        pl.kernel,
        out_type=jax.ShapeDtypeStruct((NC, N_PAD, 16), jnp.float32),
        mesh=_MESH,
        scratch_types=[
            pltpu.VMEM((nmax, CH), jnp.int32),
            pltpu.VMEM((CH, 16), jnp.float32),
            pltpu.VMEM_SHARED((N_PAD, 16), jnp.float32),
        ],
        compiler_params=_SC_PARAMS,
    )
    def k(dst_hbm, out_hbm, didx, buf, acc):
        cid = lax.axis_index("c")
        sid = lax.axis_index("s")
        start = _tile_span(cid, sid, c0, c1)
        lbase = jnp.minimum(start, TOTAL_CH - nmax)
        off = start - lbase
        pltpu.sync_copy(dst_hbm.at[pl.ds(lbase, nmax)], didx)
        _zero_buf(buf, CH, 16)
        _zero_acc(buf, acc, sid)
        _fill_ones(buf, CH, 16)
        plsc.subcore_barrier()

        def run(cnt):
            @pl.loop(0, cnt)
            def _(j):
                pltpu.sync_copy(buf, acc.at[didx.at[off + j]], add=True)

        @pl.when(cid == 0)
        def _():
            run(c0)

        @pl.when(cid == 1)
        def _():
            run(c1)

        plsc.subcore_barrier()
        _copy_out(acc, out_hbm, cid, sid)

    return k(dstc)


def _sc_agg(y, srcc, dstc, feat, c0, c1, NBUF):
    nmax = max(c0, c1)
    assert c0 % NBUF == 0 and c1 % NBUF == 0
    assert 16 * (2 * nmax * CH + NBUF * CH * feat) + N_PAD * feat <= 2097151

    @functools.partial(
        pl.kernel,
        out_type=jax.ShapeDtypeStruct((NC, N_PAD, feat), jnp.float32),
        mesh=_MESH,
        scratch_types=(
            [pltpu.VMEM((nmax, CH), jnp.int32), pltpu.VMEM((nmax, CH), jnp.int32)]
            + [pltpu.VMEM((CH, feat), jnp.float32) for _ in range(NBUF)]
            + [pltpu.VMEM_SHARED((N_PAD, feat), jnp.float32)]
            + [pltpu.SemaphoreType.DMA for _ in range(NBUF)]
        ),
        compiler_params=_SC_PARAMS,
    )
    def k(y_hbm, src_hbm, dst_hbm, out_hbm, sidx, didx, *rest):
        bufs = rest[:NBUF]
        acc = rest[NBUF]
        sems = rest[NBUF + 1:]
        cid = lax.axis_index("c")
        sid = lax.axis_index("s")
        start = _tile_span(cid, sid, c0, c1)
        lbase = jnp.minimum(start, TOTAL_CH - nmax)
        off = start - lbase
        pltpu.sync_copy(src_hbm.at[pl.ds(lbase, nmax)], sidx)
        pltpu.sync_copy(dst_hbm.at[pl.ds(lbase, nmax)], didx)
        _zero_buf(bufs[0], CH, feat)
        _zero_acc(bufs[0], acc, sid)
        plsc.subcore_barrier()

        def run(cnt):
            for b in range(NBUF):
                pltpu.make_async_copy(y_hbm.at[sidx.at[off + b]], bufs[b], sems[b]).start()

            @pl.loop(0, cnt // NBUF)
            def _(g):
                base = off + g * NBUF
                for b in range(NBUF):
                    j = base + b
                    pltpu.make_async_copy(y_hbm.at[sidx.at[j]], bufs[b], sems[b]).wait()
                    pltpu.sync_copy(bufs[b], acc.at[didx.at[j]], add=True)

                    @pl.when(j + NBUF < off + cnt)
                    def _():
                        pltpu.make_async_copy(
                            y_hbm.at[sidx.at[j + NBUF]], bufs[b], sems[b]
                        ).start()

        @pl.when(cid == 0)
        def _():
            run(c0)

        @pl.when(cid == 1)
        def _():
            run(c1)

        plsc.subcore_barrier()
        _copy_out(acc, out_hbm, cid, sid)

    return k(y, srcc, dstc)


def _dinv_from(dg_ref):
    deg = dg_ref[0, :, 0] + dg_ref[1, :, 0] + 1.0
    return lax.rsqrt(deg)


def _tc_first(x, W1, dg):
    def body(x_ref, w_ref, dg_ref, y_ref):
        dinv = _dinv_from(dg_ref)
        y_ref[...] = (
            jnp.dot(x_ref[...], w_ref[...], preferred_element_type=jnp.float32)
            * dinv[:, None]
        )

    return pl.pallas_call(
        body,
        grid=(GRID,),
        in_specs=[
            pl.BlockSpec((ROWS_BLK, 128), lambda i: (i, 0)),
            pl.BlockSpec((128, 64), lambda i: (0, 0)),
            pl.BlockSpec((NC, ROWS_BLK, 16), lambda i: (0, i, 0)),
        ],
        out_specs=pl.BlockSpec((ROWS_BLK, 64), lambda i: (i, 0)),
        out_shape=jax.ShapeDtypeStruct((N_NODES, 64), jnp.float32),
    )(x, W1, dg)


def _tc_mid(p, y_prev, dg, b, W, f_in, f_out):
    def body(p_ref, y_ref, dg_ref, b_ref, w_ref, o_ref):
        dinv = _dinv_from(dg_ref)
        h = dinv[:, None] * (p_ref[0] + p_ref[1] + y_ref[...]) + b_ref[...]
        h = jnp.maximum(h, 0.0)
        o_ref[...] = (
            jnp.dot(h, w_ref[...], preferred_element_type=jnp.float32)
            * dinv[:, None]
        )

    return pl.pallas_call(
        body,
        grid=(GRID,),
        in_specs=[
            pl.BlockSpec((NC, ROWS_BLK, f_in), lambda i: (0, i, 0)),
            pl.BlockSpec((ROWS_BLK, f_in), lambda i: (i, 0)),
            pl.BlockSpec((NC, ROWS_BLK, 16), lambda i: (0, i, 0)),
            pl.BlockSpec((1, f_in), lambda i: (0, 0)),
            pl.BlockSpec((f_in, f_out), lambda i: (0, 0)),
        ],
        out_specs=pl.BlockSpec((ROWS_BLK, f_out), lambda i: (i, 0)),
        out_shape=jax.ShapeDtypeStruct((N_NODES, f_out), jnp.float32),
    )(p, y_prev, dg, b, W)


def _tc_final(p, y3, dg, b3):
    def body(p_ref, y_ref, dg_ref, b_ref, o_ref):
        dinv = _dinv_from(dg_ref)
        o3 = (dinv[:, None] * (p_ref[0] + p_ref[1] + y_ref[...]))[:, :2] + b_ref[...]
        o_ref[...] = jax.nn.log_softmax(o3, axis=1)

    return pl.pallas_call(
        body,
        grid=(GRID,),
        in_specs=[
            pl.BlockSpec((NC, ROWS_BLK, 16), lambda i: (0, i, 0)),
            pl.BlockSpec((ROWS_BLK, 16), lambda i: (i, 0)),
            pl.BlockSpec((NC, ROWS_BLK, 16), lambda i: (0, i, 0)),
            pl.BlockSpec((1, 2), lambda i: (0, 0)),
        ],
        out_specs=pl.BlockSpec((ROWS_BLK, 2), lambda i: (i, 0)),
        out_shape=jax.ShapeDtypeStruct((N_NODES, 2), jnp.float32),
    )(p, y3, dg, b3)


SPLIT_DEG = (80, 80)
SPLIT_L1 = (136, 24)
SPLIT_L2 = (136, 24)
SPLIT_L3 = (128, 32)


def kernel(x, edge_index, W1, b1, W2, b2, W3, b3):
    pad = E_PAD - NUM_EDGES
    src = jnp.concatenate([edge_index[0], jnp.zeros((pad,), jnp.int32)])
    trash = N_NODES + (jnp.arange(pad, dtype=jnp.int32) % (N_PAD - N_NODES))
    dst = jnp.concatenate([edge_index[1], trash])
    srcc = src.reshape(TOTAL_CH, CH)
    dstc = dst.reshape(TOTAL_CH, CH)

    dg = _sc_degree(dstc, *SPLIT_DEG)
    y1 = _tc_first(x, W1, dg)
    p1 = _sc_agg(y1, srcc, dstc, 64, *SPLIT_L1, NBUF=4)
    y2 = _tc_mid(p1, y1, dg, b1.reshape(1, 64), W2, 64, 32)
    p2 = _sc_agg(y2, srcc, dstc, 32, *SPLIT_L2, NBUF=8)
    W3p = jnp.zeros((32, 16), jnp.float32).at[:, :2].set(W3)
    y3 = _tc_mid(p2, y2, dg, b2.reshape(1, 32), W3p, 32, 16)
    p3 = _sc_agg(y3, srcc, dstc, 16, *SPLIT_L3, NBUF=8)
    return _tc_final(p3, y3, dg, b3.reshape(1, 2))
